# baseline (speedup 1.0000x reference)
import functools

import jax
import jax.numpy as jnp
from jax.experimental import pallas as pl
from jax.experimental.pallas import tpu as pltpu

_BM = 2000
_BK = 1024


def _mm_kernel(a_ref, b_ref, o_ref, *, n):
    k = pl.program_id(1)

    @pl.when(k == 0)
    def _init():
        o_ref[...] = jnp.zeros_like(o_ref)

    rem = n - k * _BK
    kidx = jax.lax.broadcasted_iota(jnp.int32, (1, _BK), 1)
    a = jnp.where(kidx < rem, a_ref[...], 0.0)
    b = jnp.where(kidx.reshape(_BK, 1) < rem, b_ref[...], 0.0)
    o_ref[...] += jnp.dot(a, b, preferred_element_type=jnp.float32)


def kernel(feature, D_n_A_D_n):
    n, d = feature.shape
    m = D_n_A_D_n.shape[0]
    grid = (m // _BM, pl.cdiv(n, _BK))
    return pl.pallas_call(
        functools.partial(_mm_kernel, n=n),
        grid=grid,
        in_specs=[
            pl.BlockSpec((_BM, _BK), lambda i, k: (i, k)),
            pl.BlockSpec((_BK, d), lambda i, k: (k, 0)),
        ],
        out_specs=pl.BlockSpec((_BM, d), lambda i, k: (i, 0)),
        out_shape=jax.ShapeDtypeStruct((m, d), jnp.float32),
        compiler_params=pltpu.CompilerParams(
            dimension_semantics=("parallel", "arbitrary"),
        ),
    )(D_n_A_D_n, feature)


# speedup vs baseline: 1.0946x; 1.0946x over previous
import jax
import jax.numpy as jnp
from jax.experimental import pallas as pl
from jax.experimental.pallas import tpu as pltpu

_BM = 400


def _mm_kernel(a_ref, b_ref, o_ref):
    o_ref[...] = jnp.dot(a_ref[...], b_ref[...],
                         preferred_element_type=jnp.float32)


def kernel(feature, D_n_A_D_n):
    n, d = feature.shape
    m = D_n_A_D_n.shape[0]
    return pl.pallas_call(
        _mm_kernel,
        grid=(m // _BM,),
        in_specs=[
            pl.BlockSpec((_BM, n), lambda i: (i, 0)),
            pl.BlockSpec((n, d), lambda i: (0, 0)),
        ],
        out_specs=pl.BlockSpec((_BM, d), lambda i: (i, 0)),
        out_shape=jax.ShapeDtypeStruct((m, d), jnp.float32),
        compiler_params=pltpu.CompilerParams(
            dimension_semantics=("arbitrary",),
        ),
    )(D_n_A_D_n, feature)
